# baseline (speedup 1.0000x reference)
import functools

import jax
import jax.numpy as jnp
from jax import lax
from jax.experimental import pallas as pl
from jax.experimental.pallas import tpu as pltpu
from jax.experimental.pallas import tpu_sc as plsc

_CUTOFF = 10.0
_CUTON = 2.5
_KE = 14.399645351950548

_N_NODES = 100000
_N_EDGES = 6400000

_NC = 2
_NS = 16
_NW = _NC * _NS
_E_PER_W = _N_EDGES // _NW
_CHUNK = 2000
_N_CHUNKS = _E_PER_W // _CHUNK
_L = 16

_TC_BLK = 16384


def _gather_kernel(dist_hbm, chg_hbm, idxu_hbm, idxv_hbm, dipf_hbm,
                   r1_hbm, gx_hbm, gy_hbm, gz_hbm,
                   idxu_v, idxv_v, dist_v,
                   iu3x_v, iu3y_v, iu3z_v,
                   qi_v, qj_v, dxi_v, dyi_v, dzi_v,
                   r1_v, gx_v, gy_v, gz_v,
                   sem0, sem1, sem2, sem3, sem4):
    wid = lax.axis_index("s") * _NC + lax.axis_index("c")
    wbase = wid * _E_PER_W

    c2 = jnp.float32(_CUTOFF * _CUTOFF)
    on2 = jnp.float32(_CUTON * _CUTON)
    inv_den = jnp.float32(1.0 / (_CUTOFF**2 - _CUTON**2) ** 3)
    ke = jnp.float32(_KE)
    one = jnp.float32(1.0)
    zero = jnp.float32(0.0)
    cuton = jnp.float32(_CUTON)
    cutoff = jnp.float32(_CUTOFF)
    three = jnp.full((_L,), 3, jnp.int32)

    def chunk_body(ci, _):
        base = wbase + ci * _CHUNK
        pltpu.sync_copy(idxu_hbm.at[pl.ds(base, _CHUNK)], idxu_v)
        pltpu.sync_copy(idxv_hbm.at[pl.ds(base, _CHUNK)], idxv_v)
        pltpu.sync_copy(dist_hbm.at[pl.ds(base, _CHUNK)], dist_v)

        def idx_body(k, _):
            s = k * _L
            u3 = idxu_v[pl.ds(s, _L)] * three
            iu3x_v[pl.ds(s, _L)] = u3
            iu3y_v[pl.ds(s, _L)] = u3 + 1
            iu3z_v[pl.ds(s, _L)] = u3 + 2
            return 0

        lax.fori_loop(0, _CHUNK // _L, idx_body, 0, unroll=False)

        cp0 = pltpu.async_copy(chg_hbm.at[idxu_v], qi_v, sem0)
        cp1 = pltpu.async_copy(chg_hbm.at[idxv_v], qj_v, sem1)
        cp2 = pltpu.async_copy(dipf_hbm.at[iu3x_v], dxi_v, sem2)
        cp3 = pltpu.async_copy(dipf_hbm.at[iu3y_v], dyi_v, sem3)
        cp4 = pltpu.async_copy(dipf_hbm.at[iu3z_v], dzi_v, sem4)
        cp0.wait()
        cp1.wait()
        cp2.wait()
        cp3.wait()
        cp4.wait()

        def vec_body(k, _):
            s = k * _L
            d = dist_v[pl.ds(s, _L)]
            qi = qi_v[pl.ds(s, _L)]
            qj = qj_v[pl.ds(s, _L)]
            dx = dxi_v[pl.ds(s, _L)]
            dy = dyi_v[pl.ds(s, _L)]
            dz = dzi_v[pl.ds(s, _L)]

            chi = one / d
            chi2 = chi * chi
            d2 = d * d
            t = c2 - d2
            sw = t * t * (c2 + jnp.float32(2.0) * d2 - jnp.float32(3.0) * on2) * inv_den
            sw = jnp.where(d < cuton, one, jnp.where(d > cutoff, zero, sw))
            f = ke * sw
            r1_v[pl.ds(s, _L)] = f * qi * qj * chi
            g = f * qj * chi * chi2
            gx_v[pl.ds(s, _L)] = g * dx
            gy_v[pl.ds(s, _L)] = g * dy
            gz_v[pl.ds(s, _L)] = g * dz
            return 0

        lax.fori_loop(0, _CHUNK // _L, vec_body, 0, unroll=False)
        pltpu.sync_copy(r1_v, r1_hbm.at[pl.ds(base, _CHUNK)])
        pltpu.sync_copy(gx_v, gx_hbm.at[pl.ds(base, _CHUNK)])
        pltpu.sync_copy(gy_v, gy_hbm.at[pl.ds(base, _CHUNK)])
        pltpu.sync_copy(gz_v, gz_hbm.at[pl.ds(base, _CHUNK)])
        return 0

    lax.fori_loop(0, _N_CHUNKS, chunk_body, 0, unroll=False)


def _tc_body(vx_ref, vy_ref, vz_ref, r1_ref, gx_ref, gy_ref, gz_ref, out_ref):
    out_ref[...] = (r1_ref[...] + vx_ref[...] * gx_ref[...]
                    + vy_ref[...] * gy_ref[...] + vz_ref[...] * gz_ref[...])


@jax.jit
def _run(distances, charges, idxu, idxv, vectors, dipoles):
    dipf = dipoles.reshape(-1)
    mesh = plsc.VectorSubcoreMesh(core_axis_name="c", subcore_axis_name="s")
    edge_f32 = jax.ShapeDtypeStruct((_N_EDGES,), jnp.float32)
    sc = pl.kernel(
        _gather_kernel,
        out_type=(edge_f32, edge_f32, edge_f32, edge_f32),
        mesh=mesh,
        compiler_params=pltpu.CompilerParams(needs_layout_passes=False),
        scratch_types=[
            pltpu.VMEM((_CHUNK,), jnp.int32),
            pltpu.VMEM((_CHUNK,), jnp.int32),
            pltpu.VMEM((_CHUNK,), jnp.float32),
            pltpu.VMEM((_CHUNK,), jnp.int32),
            pltpu.VMEM((_CHUNK,), jnp.int32),
            pltpu.VMEM((_CHUNK,), jnp.int32),
            pltpu.VMEM((_CHUNK,), jnp.float32),
            pltpu.VMEM((_CHUNK,), jnp.float32),
            pltpu.VMEM((_CHUNK,), jnp.float32),
            pltpu.VMEM((_CHUNK,), jnp.float32),
            pltpu.VMEM((_CHUNK,), jnp.float32),
            pltpu.VMEM((_CHUNK,), jnp.float32),
            pltpu.VMEM((_CHUNK,), jnp.float32),
            pltpu.VMEM((_CHUNK,), jnp.float32),
            pltpu.VMEM((_CHUNK,), jnp.float32),
            pltpu.SemaphoreType.DMA,
            pltpu.SemaphoreType.DMA,
            pltpu.SemaphoreType.DMA,
            pltpu.SemaphoreType.DMA,
            pltpu.SemaphoreType.DMA,
        ],
    )
    r1, gx, gy, gz = sc(distances, charges, idxu, idxv, dipf)

    grid = (_N_EDGES // _TC_BLK,)
    spec1d = pl.BlockSpec((_TC_BLK,), lambda i: (i,))
    out = pl.pallas_call(
        _tc_body,
        grid=grid,
        in_specs=[spec1d] * 7,
        out_specs=spec1d,
        out_shape=edge_f32,
    )(vectors[:, 0], vectors[:, 1], vectors[:, 2], r1, gx, gy, gz)
    return out


def kernel(mlmm_distances, mlmm_atomic_charges, mlmm_idxu, mlmm_idxv,
           mlmm_vectors, atomic_dipoles):
    return _run(mlmm_distances, mlmm_atomic_charges, mlmm_idxu, mlmm_idxv,
                mlmm_vectors, atomic_dipoles)


# speedup vs baseline: 1.0405x; 1.0405x over previous
import functools

import jax
import jax.numpy as jnp
from jax import lax
from jax.experimental import pallas as pl
from jax.experimental.pallas import tpu as pltpu
from jax.experimental.pallas import tpu_sc as plsc

_CUTOFF = 10.0
_CUTON = 2.5
_KE = 14.399645351950548

_N_NODES = 100000
_N_EDGES = 6400000

_NC = 2
_NS = 16
_NW = _NC * _NS
_E_PER_W = _N_EDGES // _NW
_CHUNK = 2000
_N_CHUNKS = _E_PER_W // _CHUNK
_L = 16

_TC_BLK = 25600


def _gather_kernel(dist_hbm, chg_hbm, idxu_hbm, idxv_hbm, dipf_hbm,
                   r1_hbm, gx_hbm, gy_hbm, gz_hbm,
                   idxu_v, idxv_v, dist_v,
                   iu3x_v, iu3y_v, iu3z_v,
                   qi_v, qj_v, dxi_v, dyi_v, dzi_v,
                   r1_v, gx_v, gy_v, gz_v,
                   sem0, sem1, sem2, sem3, sem4):
    wid = lax.axis_index("s") * _NC + lax.axis_index("c")
    wbase = wid * _E_PER_W

    c2 = jnp.float32(_CUTOFF * _CUTOFF)
    on2 = jnp.float32(_CUTON * _CUTON)
    inv_den = jnp.float32(1.0 / (_CUTOFF**2 - _CUTON**2) ** 3)
    ke = jnp.float32(_KE)
    one = jnp.float32(1.0)
    zero = jnp.float32(0.0)
    cuton = jnp.float32(_CUTON)
    cutoff = jnp.float32(_CUTOFF)
    three = jnp.full((_L,), 3, jnp.int32)

    def chunk_body(ci, _):
        base = wbase + ci * _CHUNK
        pltpu.sync_copy(idxu_hbm.at[pl.ds(base, _CHUNK)], idxu_v)
        pltpu.sync_copy(idxv_hbm.at[pl.ds(base, _CHUNK)], idxv_v)
        pltpu.sync_copy(dist_hbm.at[pl.ds(base, _CHUNK)], dist_v)

        def idx_body(k, _):
            s = k * _L
            u3 = idxu_v[pl.ds(s, _L)] * three
            iu3x_v[pl.ds(s, _L)] = u3
            iu3y_v[pl.ds(s, _L)] = u3 + 1
            iu3z_v[pl.ds(s, _L)] = u3 + 2
            return 0

        lax.fori_loop(0, _CHUNK // _L, idx_body, 0, unroll=False)

        cp0 = pltpu.async_copy(chg_hbm.at[idxu_v], qi_v, sem0)
        cp1 = pltpu.async_copy(chg_hbm.at[idxv_v], qj_v, sem1)
        cp2 = pltpu.async_copy(dipf_hbm.at[iu3x_v], dxi_v, sem2)
        cp3 = pltpu.async_copy(dipf_hbm.at[iu3y_v], dyi_v, sem3)
        cp4 = pltpu.async_copy(dipf_hbm.at[iu3z_v], dzi_v, sem4)
        cp0.wait()
        cp1.wait()
        cp2.wait()
        cp3.wait()
        cp4.wait()

        def vec_body(k, _):
            s = k * _L
            d = dist_v[pl.ds(s, _L)]
            qi = qi_v[pl.ds(s, _L)]
            qj = qj_v[pl.ds(s, _L)]
            dx = dxi_v[pl.ds(s, _L)]
            dy = dyi_v[pl.ds(s, _L)]
            dz = dzi_v[pl.ds(s, _L)]

            chi = one / d
            chi2 = chi * chi
            d2 = d * d
            t = c2 - d2
            sw = t * t * (c2 + jnp.float32(2.0) * d2 - jnp.float32(3.0) * on2) * inv_den
            sw = jnp.where(d < cuton, one, jnp.where(d > cutoff, zero, sw))
            f = ke * sw
            r1_v[pl.ds(s, _L)] = f * qi * qj * chi
            g = f * qj * chi * chi2
            gx_v[pl.ds(s, _L)] = g * dx
            gy_v[pl.ds(s, _L)] = g * dy
            gz_v[pl.ds(s, _L)] = g * dz
            return 0

        lax.fori_loop(0, _CHUNK // _L, vec_body, 0, unroll=False)
        pltpu.sync_copy(r1_v, r1_hbm.at[pl.ds(base, _CHUNK)])
        pltpu.sync_copy(gx_v, gx_hbm.at[pl.ds(base, _CHUNK)])
        pltpu.sync_copy(gy_v, gy_hbm.at[pl.ds(base, _CHUNK)])
        pltpu.sync_copy(gz_v, gz_hbm.at[pl.ds(base, _CHUNK)])
        return 0

    lax.fori_loop(0, _N_CHUNKS, chunk_body, 0, unroll=False)


def _tc_body(vx_ref, vy_ref, vz_ref, r1_ref, gx_ref, gy_ref, gz_ref, out_ref):
    out_ref[...] = (r1_ref[...] + vx_ref[...] * gx_ref[...]
                    + vy_ref[...] * gy_ref[...] + vz_ref[...] * gz_ref[...])


@jax.jit
def _run(distances, charges, idxu, idxv, vectors, dipoles):
    dipf = dipoles.reshape(-1)
    mesh = plsc.VectorSubcoreMesh(core_axis_name="c", subcore_axis_name="s")
    edge_f32 = jax.ShapeDtypeStruct((_N_EDGES,), jnp.float32)
    sc = pl.kernel(
        _gather_kernel,
        out_type=(edge_f32, edge_f32, edge_f32, edge_f32),
        mesh=mesh,
        compiler_params=pltpu.CompilerParams(needs_layout_passes=False),
        scratch_types=[
            pltpu.VMEM((_CHUNK,), jnp.int32),
            pltpu.VMEM((_CHUNK,), jnp.int32),
            pltpu.VMEM((_CHUNK,), jnp.float32),
            pltpu.VMEM((_CHUNK,), jnp.int32),
            pltpu.VMEM((_CHUNK,), jnp.int32),
            pltpu.VMEM((_CHUNK,), jnp.int32),
            pltpu.VMEM((_CHUNK,), jnp.float32),
            pltpu.VMEM((_CHUNK,), jnp.float32),
            pltpu.VMEM((_CHUNK,), jnp.float32),
            pltpu.VMEM((_CHUNK,), jnp.float32),
            pltpu.VMEM((_CHUNK,), jnp.float32),
            pltpu.VMEM((_CHUNK,), jnp.float32),
            pltpu.VMEM((_CHUNK,), jnp.float32),
            pltpu.VMEM((_CHUNK,), jnp.float32),
            pltpu.VMEM((_CHUNK,), jnp.float32),
            pltpu.SemaphoreType.DMA,
            pltpu.SemaphoreType.DMA,
            pltpu.SemaphoreType.DMA,
            pltpu.SemaphoreType.DMA,
            pltpu.SemaphoreType.DMA,
        ],
    )
    r1, gx, gy, gz = sc(distances, charges, idxu, idxv, dipf)

    grid = (_N_EDGES // _TC_BLK,)
    spec1d = pl.BlockSpec((_TC_BLK,), lambda i: (i,))
    out = pl.pallas_call(
        _tc_body,
        grid=grid,
        in_specs=[spec1d] * 7,
        out_specs=spec1d,
        out_shape=edge_f32,
    )(vectors[:, 0], vectors[:, 1], vectors[:, 2], r1, gx, gy, gz)
    return out


def kernel(mlmm_distances, mlmm_atomic_charges, mlmm_idxu, mlmm_idxv,
           mlmm_vectors, atomic_dipoles):
    return _run(mlmm_distances, mlmm_atomic_charges, mlmm_idxu, mlmm_idxv,
                mlmm_vectors, atomic_dipoles)


# speedup vs baseline: 1.3370x; 1.2850x over previous
import functools

import jax
import jax.numpy as jnp
from jax import lax
from jax.experimental import pallas as pl
from jax.experimental.pallas import tpu as pltpu
from jax.experimental.pallas import tpu_sc as plsc

_CUTOFF = 10.0
_CUTON = 2.5
_KE = 14.399645351950548

_N_NODES = 100000
_N_EDGES = 6400000

_NC = 2
_NS = 16
_NW = _NC * _NS
_E_PER_W = _N_EDGES // _NW
_CHUNK = 2000
_N_CHUNKS = _E_PER_W // _CHUNK
_L = 16

_TC_BLK = 25600


def _gather_kernel(dist_hbm, chg_hbm, idxu_hbm, idxv_hbm, dipf_hbm,
                   r1_hbm, gx_hbm, gy_hbm, gz_hbm,
                   chg_v,
                   idxu_v, idxv_v, dist_v,
                   iu3x_v, iu3y_v, iu3z_v,
                   dxi_v, dyi_v, dzi_v,
                   r1_v, gx_v, gy_v, gz_v,
                   sem2, sem3, sem4):
    wid = lax.axis_index("s") * _NC + lax.axis_index("c")
    wbase = wid * _E_PER_W
    pltpu.sync_copy(chg_hbm, chg_v)

    c2 = jnp.float32(_CUTOFF * _CUTOFF)
    on2 = jnp.float32(_CUTON * _CUTON)
    inv_den = jnp.float32(1.0 / (_CUTOFF**2 - _CUTON**2) ** 3)
    ke = jnp.float32(_KE)
    one = jnp.float32(1.0)
    zero = jnp.float32(0.0)
    cuton = jnp.float32(_CUTON)
    cutoff = jnp.float32(_CUTOFF)
    three = jnp.full((_L,), 3, jnp.int32)

    def chunk_body(ci, _):
        base = wbase + ci * _CHUNK
        pltpu.sync_copy(idxu_hbm.at[pl.ds(base, _CHUNK)], idxu_v)
        pltpu.sync_copy(idxv_hbm.at[pl.ds(base, _CHUNK)], idxv_v)
        pltpu.sync_copy(dist_hbm.at[pl.ds(base, _CHUNK)], dist_v)

        def idx_body(k, _):
            s = k * _L
            u3 = idxu_v[pl.ds(s, _L)] * three
            iu3x_v[pl.ds(s, _L)] = u3
            iu3y_v[pl.ds(s, _L)] = u3 + 1
            iu3z_v[pl.ds(s, _L)] = u3 + 2
            return 0

        lax.fori_loop(0, _CHUNK // _L, idx_body, 0, unroll=False)

        cp2 = pltpu.async_copy(dipf_hbm.at[iu3x_v], dxi_v, sem2)
        cp3 = pltpu.async_copy(dipf_hbm.at[iu3y_v], dyi_v, sem3)
        cp4 = pltpu.async_copy(dipf_hbm.at[iu3z_v], dzi_v, sem4)
        cp2.wait()
        cp3.wait()
        cp4.wait()

        def vec_body(k, _):
            s = k * _L
            d = dist_v[pl.ds(s, _L)]
            qi = plsc.load_gather(chg_v, [idxu_v[pl.ds(s, _L)]])
            qj = plsc.load_gather(chg_v, [idxv_v[pl.ds(s, _L)]])
            dx = dxi_v[pl.ds(s, _L)]
            dy = dyi_v[pl.ds(s, _L)]
            dz = dzi_v[pl.ds(s, _L)]

            chi = one / d
            chi2 = chi * chi
            d2 = d * d
            t = c2 - d2
            sw = t * t * (c2 + jnp.float32(2.0) * d2 - jnp.float32(3.0) * on2) * inv_den
            sw = jnp.where(d < cuton, one, jnp.where(d > cutoff, zero, sw))
            f = ke * sw
            r1_v[pl.ds(s, _L)] = f * qi * qj * chi
            g = f * qj * chi * chi2
            gx_v[pl.ds(s, _L)] = g * dx
            gy_v[pl.ds(s, _L)] = g * dy
            gz_v[pl.ds(s, _L)] = g * dz
            return 0

        lax.fori_loop(0, _CHUNK // _L, vec_body, 0, unroll=False)
        pltpu.sync_copy(r1_v, r1_hbm.at[pl.ds(base, _CHUNK)])
        pltpu.sync_copy(gx_v, gx_hbm.at[pl.ds(base, _CHUNK)])
        pltpu.sync_copy(gy_v, gy_hbm.at[pl.ds(base, _CHUNK)])
        pltpu.sync_copy(gz_v, gz_hbm.at[pl.ds(base, _CHUNK)])
        return 0

    lax.fori_loop(0, _N_CHUNKS, chunk_body, 0, unroll=False)


def _tc_body(vx_ref, vy_ref, vz_ref, r1_ref, gx_ref, gy_ref, gz_ref, out_ref):
    out_ref[...] = (r1_ref[...] + vx_ref[...] * gx_ref[...]
                    + vy_ref[...] * gy_ref[...] + vz_ref[...] * gz_ref[...])


@jax.jit
def _run(distances, charges, idxu, idxv, vectors, dipoles):
    dipf = dipoles.reshape(-1)
    mesh = plsc.VectorSubcoreMesh(core_axis_name="c", subcore_axis_name="s")
    edge_f32 = jax.ShapeDtypeStruct((_N_EDGES,), jnp.float32)
    sc = pl.kernel(
        _gather_kernel,
        out_type=(edge_f32, edge_f32, edge_f32, edge_f32),
        mesh=mesh,
        compiler_params=pltpu.CompilerParams(needs_layout_passes=False),
        scratch_types=[
            pltpu.VMEM((_N_NODES,), jnp.float32),
            pltpu.VMEM((_CHUNK,), jnp.int32),
            pltpu.VMEM((_CHUNK,), jnp.int32),
            pltpu.VMEM((_CHUNK,), jnp.float32),
            pltpu.VMEM((_CHUNK,), jnp.int32),
            pltpu.VMEM((_CHUNK,), jnp.int32),
            pltpu.VMEM((_CHUNK,), jnp.int32),
            pltpu.VMEM((_CHUNK,), jnp.float32),
            pltpu.VMEM((_CHUNK,), jnp.float32),
            pltpu.VMEM((_CHUNK,), jnp.float32),
            pltpu.VMEM((_CHUNK,), jnp.float32),
            pltpu.VMEM((_CHUNK,), jnp.float32),
            pltpu.VMEM((_CHUNK,), jnp.float32),
            pltpu.VMEM((_CHUNK,), jnp.float32),
            pltpu.SemaphoreType.DMA,
            pltpu.SemaphoreType.DMA,
            pltpu.SemaphoreType.DMA,
        ],
    )
    r1, gx, gy, gz = sc(distances, charges, idxu, idxv, dipf)

    grid = (_N_EDGES // _TC_BLK,)
    spec1d = pl.BlockSpec((_TC_BLK,), lambda i: (i,))
    out = pl.pallas_call(
        _tc_body,
        grid=grid,
        in_specs=[spec1d] * 7,
        out_specs=spec1d,
        out_shape=edge_f32,
    )(vectors[:, 0], vectors[:, 1], vectors[:, 2], r1, gx, gy, gz)
    return out


def kernel(mlmm_distances, mlmm_atomic_charges, mlmm_idxu, mlmm_idxv,
           mlmm_vectors, atomic_dipoles):
    return _run(mlmm_distances, mlmm_atomic_charges, mlmm_idxu, mlmm_idxv,
                mlmm_vectors, atomic_dipoles)


# speedup vs baseline: 1.3821x; 1.0337x over previous
import functools

import jax
import jax.numpy as jnp
from jax import lax
from jax.experimental import pallas as pl
from jax.experimental.pallas import tpu as pltpu
from jax.experimental.pallas import tpu_sc as plsc

_CUTOFF = 10.0
_CUTON = 2.5
_KE = 14.399645351950548

_N_NODES = 100000
_N_EDGES = 6400000

_NC = 2
_NS = 16
_NW = _NC * _NS
_E_PER_W = _N_EDGES // _NW
_CHUNK = 2000
_N_CHUNKS = _E_PER_W // _CHUNK
_L = 16

_TC_BLK = 25600


def _gather_kernel(dist_hbm, chg_hbm, idxu_hbm, idxv_hbm, dipf_hbm,
                   r1_hbm, gx_hbm, gy_hbm, gz_hbm,
                   chg_v,
                   idxu_v, idxv_v, dist_v,
                   iu3x_v, iu3y_v, iu3z_v,
                   dxi_v, dyi_v, dzi_v,
                   r1_v, gx_v, gy_v, gz_v,
                   sem2, sem3, sem4):
    wid = lax.axis_index("s") * _NC + lax.axis_index("c")
    wbase = wid * _E_PER_W
    pltpu.sync_copy(chg_hbm, chg_v)

    c2 = jnp.float32(_CUTOFF * _CUTOFF)
    on2 = jnp.float32(_CUTON * _CUTON)
    inv_den = jnp.float32(1.0 / (_CUTOFF**2 - _CUTON**2) ** 3)
    ke = jnp.float32(_KE)
    one = jnp.float32(1.0)
    zero = jnp.float32(0.0)
    cuton = jnp.float32(_CUTON)
    cutoff = jnp.float32(_CUTOFF)
    three = jnp.full((_L,), 3, jnp.int32)

    def chunk_body(ci, _):
        base = wbase + ci * _CHUNK
        pltpu.sync_copy(idxu_hbm.at[pl.ds(base, _CHUNK)], idxu_v)
        pltpu.sync_copy(idxv_hbm.at[pl.ds(base, _CHUNK)], idxv_v)
        pltpu.sync_copy(dist_hbm.at[pl.ds(base, _CHUNK)], dist_v)

        @plsc.parallel_loop(0, _CHUNK // _L, unroll=4)
        def idx_body(k):
            s = k * _L
            u3 = idxu_v[pl.ds(s, _L)] * three
            iu3x_v[pl.ds(s, _L)] = u3
            iu3y_v[pl.ds(s, _L)] = u3 + 1
            iu3z_v[pl.ds(s, _L)] = u3 + 2

        cp2 = pltpu.async_copy(dipf_hbm.at[iu3x_v], dxi_v, sem2)
        cp3 = pltpu.async_copy(dipf_hbm.at[iu3y_v], dyi_v, sem3)
        cp4 = pltpu.async_copy(dipf_hbm.at[iu3z_v], dzi_v, sem4)
        cp2.wait()
        cp3.wait()
        cp4.wait()

        @plsc.parallel_loop(0, _CHUNK // _L, unroll=4)
        def vec_body(k):
            s = k * _L
            d = dist_v[pl.ds(s, _L)]
            qi = plsc.load_gather(chg_v, [idxu_v[pl.ds(s, _L)]])
            qj = plsc.load_gather(chg_v, [idxv_v[pl.ds(s, _L)]])
            dx = dxi_v[pl.ds(s, _L)]
            dy = dyi_v[pl.ds(s, _L)]
            dz = dzi_v[pl.ds(s, _L)]

            chi = one / d
            chi2 = chi * chi
            d2 = d * d
            t = c2 - d2
            sw = t * t * (c2 + jnp.float32(2.0) * d2 - jnp.float32(3.0) * on2) * inv_den
            sw = jnp.where(d < cuton, one, jnp.where(d > cutoff, zero, sw))
            f = ke * sw
            r1_v[pl.ds(s, _L)] = f * qi * qj * chi
            g = f * qj * chi * chi2
            gx_v[pl.ds(s, _L)] = g * dx
            gy_v[pl.ds(s, _L)] = g * dy
            gz_v[pl.ds(s, _L)] = g * dz

        pltpu.sync_copy(r1_v, r1_hbm.at[pl.ds(base, _CHUNK)])
        pltpu.sync_copy(gx_v, gx_hbm.at[pl.ds(base, _CHUNK)])
        pltpu.sync_copy(gy_v, gy_hbm.at[pl.ds(base, _CHUNK)])
        pltpu.sync_copy(gz_v, gz_hbm.at[pl.ds(base, _CHUNK)])
        return 0

    lax.fori_loop(0, _N_CHUNKS, chunk_body, 0, unroll=False)


def _tc_body(vx_ref, vy_ref, vz_ref, r1_ref, gx_ref, gy_ref, gz_ref, out_ref):
    out_ref[...] = (r1_ref[...] + vx_ref[...] * gx_ref[...]
                    + vy_ref[...] * gy_ref[...] + vz_ref[...] * gz_ref[...])


@jax.jit
def _run(distances, charges, idxu, idxv, vectors, dipoles):
    dipf = dipoles.reshape(-1)
    mesh = plsc.VectorSubcoreMesh(core_axis_name="c", subcore_axis_name="s")
    edge_f32 = jax.ShapeDtypeStruct((_N_EDGES,), jnp.float32)
    sc = pl.kernel(
        _gather_kernel,
        out_type=(edge_f32, edge_f32, edge_f32, edge_f32),
        mesh=mesh,
        compiler_params=pltpu.CompilerParams(needs_layout_passes=False),
        scratch_types=[
            pltpu.VMEM((_N_NODES,), jnp.float32),
            pltpu.VMEM((_CHUNK,), jnp.int32),
            pltpu.VMEM((_CHUNK,), jnp.int32),
            pltpu.VMEM((_CHUNK,), jnp.float32),
            pltpu.VMEM((_CHUNK,), jnp.int32),
            pltpu.VMEM((_CHUNK,), jnp.int32),
            pltpu.VMEM((_CHUNK,), jnp.int32),
            pltpu.VMEM((_CHUNK,), jnp.float32),
            pltpu.VMEM((_CHUNK,), jnp.float32),
            pltpu.VMEM((_CHUNK,), jnp.float32),
            pltpu.VMEM((_CHUNK,), jnp.float32),
            pltpu.VMEM((_CHUNK,), jnp.float32),
            pltpu.VMEM((_CHUNK,), jnp.float32),
            pltpu.VMEM((_CHUNK,), jnp.float32),
            pltpu.SemaphoreType.DMA,
            pltpu.SemaphoreType.DMA,
            pltpu.SemaphoreType.DMA,
        ],
    )
    r1, gx, gy, gz = sc(distances, charges, idxu, idxv, dipf)

    grid = (_N_EDGES // _TC_BLK,)
    spec1d = pl.BlockSpec((_TC_BLK,), lambda i: (i,))
    out = pl.pallas_call(
        _tc_body,
        grid=grid,
        in_specs=[spec1d] * 7,
        out_specs=spec1d,
        out_shape=edge_f32,
    )(vectors[:, 0], vectors[:, 1], vectors[:, 2], r1, gx, gy, gz)
    return out


def kernel(mlmm_distances, mlmm_atomic_charges, mlmm_idxu, mlmm_idxv,
           mlmm_vectors, atomic_dipoles):
    return _run(mlmm_distances, mlmm_atomic_charges, mlmm_idxu, mlmm_idxv,
                mlmm_vectors, atomic_dipoles)
